# baseline (speedup 1.0000x reference)
import functools

import jax
import jax.numpy as jnp
from jax.experimental import pallas as pl
from jax.experimental.pallas import tpu as pltpu

B = 64
N_SAMPLES = 64000
F = 512
STRIDE = 256
N_FRAMES = (N_SAMPLES - F) // STRIDE + 1
N_CHUNKS = N_SAMPLES // STRIDE


def _tc_block(in_ref, out_ref):
    out_ref[0, 0, :, 0:STRIDE] = in_ref[0, 0:N_FRAMES, :]
    out_ref[0, 0, :, STRIDE:F] = in_ref[0, 1 : N_FRAMES + 1, :]


_tc_kernel = pl.pallas_call(
    _tc_block,
    grid=(B,),
    in_specs=[pl.BlockSpec((1, N_CHUNKS, STRIDE), lambda b: (b, 0, 0))],
    out_specs=pl.BlockSpec((1, 1, N_FRAMES, F), lambda b: (b, 0, 0, 0)),
    out_shape=jax.ShapeDtypeStruct((B, 1, N_FRAMES, F), jnp.float32),
)


def kernel(sig):
    return _tc_kernel(sig.reshape(B, N_CHUNKS, STRIDE))


# speedup vs baseline: 4.1207x; 4.1207x over previous
import functools

import jax
import jax.numpy as jnp
from jax import lax
from jax.experimental import pallas as pl
from jax.experimental.pallas import tpu as pltpu
from jax.experimental.pallas import tpu_sc as plsc

B = 64
N_SAMPLES = 64000
F = 512
STRIDE = 256
N_FRAMES = (N_SAMPLES - F) // STRIDE + 1
N_CHUNKS = N_SAMPLES // STRIDE

NUM_CORES = 2
NUM_SUBCORES = 16
NUM_WORKERS = NUM_CORES * NUM_SUBCORES
ROWS_PER_WORKER = B // NUM_WORKERS

_mesh = plsc.VectorSubcoreMesh(core_axis_name="c", subcore_axis_name="s")


@functools.partial(
    pl.kernel,
    mesh=_mesh,
    out_type=jax.ShapeDtypeStruct((B, 1, N_FRAMES, F), jnp.float32),
    scratch_types=[
        pltpu.VMEM((N_CHUNKS, STRIDE), jnp.float32),
        pltpu.VMEM((N_CHUNKS, STRIDE), jnp.float32),
        pltpu.SemaphoreType.DMA,
        pltpu.SemaphoreType.DMA,
        pltpu.SemaphoreType.DMA,
    ],
    compiler_params=pltpu.CompilerParams(use_tc_tiling_on_sc=False),
)
def _frames_kernel(sig_hbm, out_hbm, buf0, buf1, sem_in, sem_o0, sem_o1):
    wid = lax.axis_index("s") * NUM_CORES + lax.axis_index("c")
    bufs = (buf0, buf1)
    out_sems = (sem_o0, sem_o1)
    stores = []
    for r in range(ROWS_PER_WORKER):
        b = wid * ROWS_PER_WORKER + r
        buf = bufs[r % 2]
        pltpu.async_copy(sig_hbm.at[b], buf, sem_in).wait()
        stores.append(
            pltpu.async_copy(
                buf.at[pl.ds(0, N_FRAMES)],
                out_hbm.at[b, 0, :, pl.ds(0, STRIDE)],
                out_sems[r % 2],
            )
        )
        stores.append(
            pltpu.async_copy(
                buf.at[pl.ds(1, N_FRAMES)],
                out_hbm.at[b, 0, :, pl.ds(STRIDE, STRIDE)],
                out_sems[r % 2],
            )
        )
    for cp in stores:
        cp.wait()


def kernel(sig):
    return _frames_kernel(sig.reshape(B, N_CHUNKS, STRIDE))


# speedup vs baseline: 4.1335x; 1.0031x over previous
import functools

import jax
import jax.numpy as jnp
from jax import lax
from jax.experimental import pallas as pl
from jax.experimental.pallas import tpu as pltpu
from jax.experimental.pallas import tpu_sc as plsc

B = 64
N_SAMPLES = 64000
F = 512
STRIDE = 256
N_FRAMES = (N_SAMPLES - F) // STRIDE + 1
N_CHUNKS = N_SAMPLES // STRIDE

NUM_CORES = 2
NUM_SUBCORES = 16
NUM_WORKERS = NUM_CORES * NUM_SUBCORES
ROWS_PER_WORKER = B // NUM_WORKERS

_mesh = plsc.VectorSubcoreMesh(core_axis_name="c", subcore_axis_name="s")


@functools.partial(
    pl.kernel,
    mesh=_mesh,
    out_type=jax.ShapeDtypeStruct((B, 1, N_FRAMES, F), jnp.float32),
    scratch_types=[
        pltpu.VMEM((ROWS_PER_WORKER, N_CHUNKS, STRIDE), jnp.float32),
        pltpu.SemaphoreType.DMA,
        pltpu.SemaphoreType.DMA,
    ],
    compiler_params=pltpu.CompilerParams(use_tc_tiling_on_sc=False),
)
def _frames_kernel(sig_hbm, out_hbm, buf, sem_in, sem_out):
    wid = lax.axis_index("s") * NUM_CORES + lax.axis_index("c")
    b0 = wid * ROWS_PER_WORKER
    pltpu.async_copy(sig_hbm.at[pl.ds(b0, ROWS_PER_WORKER)], buf, sem_in).wait()
    lo = pltpu.async_copy(
        buf.at[:, pl.ds(0, N_FRAMES)],
        out_hbm.at[pl.ds(b0, ROWS_PER_WORKER), 0, :, pl.ds(0, STRIDE)],
        sem_out,
    )
    hi = pltpu.async_copy(
        buf.at[:, pl.ds(1, N_FRAMES)],
        out_hbm.at[pl.ds(b0, ROWS_PER_WORKER), 0, :, pl.ds(STRIDE, STRIDE)],
        sem_out,
    )
    lo.wait()
    hi.wait()


def kernel(sig):
    return _frames_kernel(sig.reshape(B, N_CHUNKS, STRIDE))
